# baseline (speedup 1.0000x reference)
import functools

import jax
import jax.numpy as jnp
from jax.experimental import pallas as pl
from jax.experimental.pallas import tpu as pltpu

_IMG = 224.0
_CROP = 7.0
_ROI = 2
_A = _ROI / _CROP


def _axis_weight_mats(coord_row, dimsize, L):
    lmn = coord_row / _IMG * _CROP
    t = -1.0 + 2.0 * lmn / _CROP
    iota = jax.lax.broadcasted_iota(jnp.int32, (dimsize, L), 0)
    mats = []
    for p in range(_ROI):
        base = (2.0 * p + 1.0) / _ROI - 1.0
        g = _A * base + t
        pos = jnp.clip(((g + 1.0) * dimsize - 1.0) * 0.5, 0.0, dimsize - 1.0)
        p0f = jnp.floor(pos)
        frac = pos - p0f
        p0 = p0f.astype(jnp.int32)
        p1 = jnp.minimum(p0 + 1, dimsize - 1)
        m = jnp.where(iota == p0, 1.0 - frac, 0.0) + jnp.where(
            iota == p1, frac, 0.0
        )
        mats.append(m)
    return mats


def _pool_kernel(lmx_ref, lmy_ref, feat_ref, out_ref, *, C, H, W, L):
    featCH = feat_ref[0]
    lx = lmx_ref[0]
    ly = lmy_ref[0]

    Wx0, Wx1 = _axis_weight_mats(lx, W, L)
    Wy0, Wy1 = _axis_weight_mats(ly, H, L)

    Wx = jnp.concatenate([Wx0, Wx1], axis=1)
    tmp = jnp.dot(featCH, Wx, preferred_element_type=jnp.float32)
    tmp3 = tmp.reshape(C, H, 2 * L)

    Wy0r = jnp.concatenate([Wy0, Wy0], axis=1)[None]
    Wy1r = jnp.concatenate([Wy1, Wy1], axis=1)[None]
    v0 = jnp.sum(tmp3 * Wy0r, axis=1)
    v1 = jnp.sum(tmp3 * Wy1r, axis=1)
    vm = jnp.maximum(v0, v1)
    pooled = jnp.maximum(vm[:, :L], vm[:, L:])
    out_ref[0] = pooled.T


def _mm_kernel(x_ref, w_ref, b_ref, out_ref):
    acc = jax.lax.dot_general(
        x_ref[...],
        w_ref[...],
        (((1,), (1,)), ((), ())),
        preferred_element_type=jnp.float32,
    )
    out_ref[...] = jnp.maximum(acc + b_ref[...], 0.0)


def kernel(features, landmarks, W_lin, b_lin):
    B, C, H, W = features.shape
    L = landmarks.shape[1] // 2
    OUT, K = W_lin.shape

    featCH = features.reshape(B, C * H, W)
    lmx = landmarks[:, 0::2].reshape(B, 1, L)
    lmy = landmarks[:, 1::2].reshape(B, 1, L)

    pooled = pl.pallas_call(
        functools.partial(_pool_kernel, C=C, H=H, W=W, L=L),
        grid=(B,),
        in_specs=[
            pl.BlockSpec((1, 1, L), lambda b: (b, 0, 0)),
            pl.BlockSpec((1, 1, L), lambda b: (b, 0, 0)),
            pl.BlockSpec((1, C * H, W), lambda b: (b, 0, 0)),
        ],
        out_specs=pl.BlockSpec((1, L, C), lambda b: (b, 0, 0)),
        out_shape=jax.ShapeDtypeStruct((B, L, C), jnp.float32),
        compiler_params=pltpu.CompilerParams(
            dimension_semantics=("parallel",),
        ),
        name="roi_pool",
    )(lmx, lmy, featCH)

    flat = pooled.reshape(B, L * C)
    NB = 512
    b2 = b_lin.reshape(1, OUT)
    out = pl.pallas_call(
        _mm_kernel,
        grid=(OUT // NB,),
        in_specs=[
            pl.BlockSpec((B, K), lambda j: (0, 0)),
            pl.BlockSpec((NB, K), lambda j: (j, 0)),
            pl.BlockSpec((1, NB), lambda j: (0, j)),
        ],
        out_specs=pl.BlockSpec((B, NB), lambda j: (0, j)),
        out_shape=jax.ShapeDtypeStruct((B, OUT), jnp.float32),
        compiler_params=pltpu.CompilerParams(
            dimension_semantics=("parallel",),
        ),
        name="linear_relu",
    )(flat, W_lin, b2)
    return out
